# Initial kernel scaffold; baseline (speedup 1.0000x reference)
import jax
import jax.numpy as jnp
from jax.experimental import pallas as pl


def kernel(teacher_boxes, teacher_scores, student_boxes, student_scores):
    raise NotImplementedError("write your pallas kernel here")


# speedup vs baseline: 1.1597x; 1.1597x over previous
import jax
import jax.numpy as jnp
from jax.experimental import pallas as pl
from jax.experimental.pallas import tpu as pltpu

N = 5000
NP = 5120
TBLK = 512
SBLK = 1280


def _tile_kernel(t_ref, s_ref, out_ref, tot_ref, cnt_ref):
    i = pl.program_id(0)
    j = pl.program_id(1)
    ni = pl.num_programs(0)
    nj = pl.num_programs(1)

    @pl.when((i == 0) & (j == 0))
    def _init():
        tot_ref[...] = jnp.zeros_like(tot_ref)
        cnt_ref[...] = jnp.zeros_like(cnt_ref)

    t = t_ref[...]
    s = s_ref[...]

    tx1 = t[:, 0:1]
    ty1 = t[:, 1:2]
    tx2 = t[:, 2:3]
    ty2 = t[:, 3:4]
    sx1 = s[0:1, :]
    sy1 = s[1:2, :]
    sx2 = s[2:3, :]
    sy2 = s[3:4, :]

    iw = jnp.maximum(jnp.minimum(tx2, sx2) - jnp.maximum(tx1, sx1), 0.0)
    ih = jnp.maximum(jnp.minimum(ty2, sy2) - jnp.maximum(ty1, sy1), 0.0)
    inter = iw * ih

    tarea = (tx2 - tx1) * (ty2 - ty1)
    sarea = (sx2 - sx1) * (sy2 - sy1)
    union = (tarea + sarea) - inter

    mask = ((inter + inter) >= union) & (union > 0.0)

    tc = t - 0.5
    sc = s - 0.5
    ta2 = jnp.sum(tc * tc, axis=1, keepdims=True)
    sa2 = jnp.sum(sc * sc, axis=0, keepdims=True)
    d = jax.lax.dot_general(
        tc, sc, (((1,), (0,)), ((), ())), preferred_element_type=jnp.float32
    )
    sq = (ta2 + sa2) - (d + d)

    tot_ref[...] += jnp.sum(jnp.where(mask, sq, 0.0), keepdims=True)
    cnt_ref[...] += jnp.sum(mask.astype(jnp.float32), keepdims=True)

    @pl.when((i == ni - 1) & (j == nj - 1))
    def _finish():
        tot = tot_ref[...]
        cnt = cnt_ref[...]
        out_ref[...] = jnp.where(
            cnt > 0.0, tot / (4.0 * jnp.maximum(cnt, 1.0)), 0.0
        )


def _pad_boxes(boxes, sentinel):
    pad = jnp.tile(jnp.asarray(sentinel, jnp.float32)[None, :], (NP - N, 1))
    return jnp.concatenate([boxes.astype(jnp.float32), pad], axis=0)


def kernel(teacher_boxes, teacher_scores, student_boxes, student_scores):
    del teacher_scores, student_scores
    t = _pad_boxes(teacher_boxes, [-8.0, -8.0, -7.0, -7.0])
    s = _pad_boxes(student_boxes, [8.0, 8.0, 9.0, 9.0]).T

    out = pl.pallas_call(
        _tile_kernel,
        grid=(NP // TBLK, NP // SBLK),
        in_specs=[
            pl.BlockSpec((TBLK, 4), lambda i, j: (i, 0)),
            pl.BlockSpec((4, SBLK), lambda i, j: (0, j)),
        ],
        out_specs=pl.BlockSpec((1, 1), lambda i, j: (0, 0)),
        out_shape=jax.ShapeDtypeStruct((1, 1), jnp.float32),
        scratch_shapes=[
            pltpu.VMEM((1, 1), jnp.float32),
            pltpu.VMEM((1, 1), jnp.float32),
        ],
    )(t, s)
    return out[0, 0]
